# Initial kernel scaffold; baseline (speedup 1.0000x reference)
import jax
import jax.numpy as jnp
from jax.experimental import pallas as pl


def kernel(x, edge_index, edge_attr, Wself1, Wneigh1, b1, Wself2, Wneigh2, b2, Wself3, Wneigh3, b3, Wa, ba, W2a, b2a, Wb, bb, W2b, b2b):
    raise NotImplementedError("write your pallas kernel here")


# speedup vs baseline: 3.7557x; 3.7557x over previous
import functools

import jax
import jax.numpy as jnp
from jax import lax
from jax.experimental import pallas as pl
from jax.experimental.pallas import tpu as pltpu
from jax.experimental.pallas import tpu_sc as plsc

_N = 10000
_E = 320000
_D = 128
_DH = _D // 2
_NC = 2
_NS = 16
_B = 128
_NB = 160
_NP = 10240
_RPT = _NP // _NS
_DUMMY = _N
_EPAD = _NS * _NB * _B
_BLK = 1024
_G = _NP // _BLK

_mesh = plsc.VectorSubcoreMesh(core_axis_name="c", subcore_axis_name="s")
_SC_PARAMS = pltpu.CompilerParams(use_tc_tiling_on_sc=False)


def _zero_rows(ref, nrows, ncols16):
    z16 = jnp.zeros((16,), jnp.float32)

    def _row(i, carry):
        for j in range(ncols16):
            ref[i, pl.ds(j * 16, 16)] = z16
        return carry

    lax.fori_loop(0, nrows, _row, 0)


def _sc_common(c, s, hf_hbm, srci_hbm, dsti_hbm, acc_hbm,
               sidx, didx, buf0, buf1, acc_sh, sem0, sem1):
    pltpu.sync_copy(srci_hbm.at[c].at[s], sidx)
    pltpu.sync_copy(dsti_hbm.at[s], didx)

    _zero_rows(buf0, _B, _DH // 16)
    base = s * _RPT

    def _zacc(k, carry):
        pltpu.sync_copy(buf0, acc_sh.at[pl.ds(base + k * _B, _B)])
        return carry

    lax.fori_loop(0, _RPT // _B, _zacc, 0)
    plsc.subcore_barrier()

    pltpu.async_copy(hf_hbm.at[sidx.at[0]], buf0, sem0)

    def _main(g, carry):
        b0 = 2 * g
        pltpu.async_copy(hf_hbm.at[sidx.at[b0 + 1]], buf1, sem1)
        pltpu.make_async_copy(hf_hbm.at[sidx.at[b0]], buf0, sem0).wait()
        pltpu.sync_copy(buf0, acc_sh.at[didx.at[b0]], add=True)

        @pl.when(g < _NB // 2 - 1)
        def _():
            pltpu.async_copy(hf_hbm.at[sidx.at[b0 + 2]], buf0, sem0)

        pltpu.make_async_copy(hf_hbm.at[sidx.at[b0 + 1]], buf1, sem1).wait()
        pltpu.sync_copy(buf1, acc_sh.at[didx.at[b0 + 1]], add=True)
        return carry

    lax.fori_loop(0, _NB // 2, _main, 0)
    plsc.subcore_barrier()

    def _wout(k, carry):
        r = base + k * _B
        pltpu.sync_copy(acc_sh.at[pl.ds(r, _B)], buf0)
        pltpu.sync_copy(buf0, acc_hbm.at[c].at[pl.ds(r, _B)])
        return carry

    lax.fori_loop(0, _RPT // _B, _wout, 0)


@functools.partial(
    pl.kernel,
    out_type=(
        jax.ShapeDtypeStruct((_NC, _NP, _DH), jnp.float32),
        jax.ShapeDtypeStruct((_NP, 16), jnp.float32),
    ),
    mesh=_mesh,
    scratch_types=[
        pltpu.VMEM((_NB, _B), jnp.int32),
        pltpu.VMEM((_NB, _B), jnp.int32),
        pltpu.VMEM((_B, _DH), jnp.float32),
        pltpu.VMEM((_B, _DH), jnp.float32),
        pltpu.VMEM((_B, 16), jnp.float32),
        pltpu.VMEM((_RPT, 16), jnp.float32),
        pltpu.VMEM_SHARED((_NP, _DH), jnp.float32),
        pltpu.VMEM_SHARED((_NP, 16), jnp.float32),
        pltpu.SemaphoreType.DMA,
        pltpu.SemaphoreType.DMA,
    ],
    compiler_params=_SC_PARAMS,
)
def _sc_agg_deg(hf_hbm, srci_hbm, dsti_hbm, acc_hbm, deg_hbm,
                sidx, didx, buf0, buf1, ones, zd, acc_sh, deg_sh,
                sem0, sem1):
    c = lax.axis_index("c")
    s = lax.axis_index("s")

    @pl.when(c == 0)
    def _():
        _zero_rows(zd, _RPT, 1)
        o16 = jnp.ones((16,), jnp.float32)

        def _orow(i, carry):
            ones[i, :] = o16
            return carry

        lax.fori_loop(0, _B, _orow, 0)
        pltpu.sync_copy(zd, deg_sh.at[pl.ds(s * _RPT, _RPT)])

    _sc_common(c, s, hf_hbm, srci_hbm, dsti_hbm, acc_hbm,
               sidx, didx, buf0, buf1, acc_sh, sem0, sem1)

    @pl.when(c == 0)
    def _():
        def _degb(b, carry):
            pltpu.sync_copy(ones, deg_sh.at[didx.at[b]], add=True)
            return carry

        lax.fori_loop(0, _NB, _degb, 0)
        plsc.subcore_barrier()
        pltpu.sync_copy(deg_sh.at[pl.ds(s * _RPT, _RPT)], zd)
        pltpu.sync_copy(zd, deg_hbm.at[pl.ds(s * _RPT, _RPT)])


@functools.partial(
    pl.kernel,
    out_type=jax.ShapeDtypeStruct((_NC, _NP, _DH), jnp.float32),
    mesh=_mesh,
    scratch_types=[
        pltpu.VMEM((_NB, _B), jnp.int32),
        pltpu.VMEM((_NB, _B), jnp.int32),
        pltpu.VMEM((_B, _DH), jnp.float32),
        pltpu.VMEM((_B, _DH), jnp.float32),
        pltpu.VMEM_SHARED((_NP, _DH), jnp.float32),
        pltpu.SemaphoreType.DMA,
        pltpu.SemaphoreType.DMA,
    ],
    compiler_params=_SC_PARAMS,
)
def _sc_agg(hf_hbm, srci_hbm, dsti_hbm, acc_hbm,
            sidx, didx, buf0, buf1, acc_sh, sem0, sem1):
    c = lax.axis_index("c")
    s = lax.axis_index("s")
    _sc_common(c, s, hf_hbm, srci_hbm, dsti_hbm, acc_hbm,
               sidx, didx, buf0, buf1, acc_sh, sem0, sem1)


def _tc1_body(h_ref, acc_ref, deg_ref, ws_ref, wn_ref, b_ref, out_ref, rd_ref):
    a = jnp.concatenate([acc_ref[0], acc_ref[1]], axis=-1)
    rd = 1.0 / jnp.maximum(deg_ref[:, 0:1], 1.0)
    mean = a * rd
    out = (jnp.dot(h_ref[...], ws_ref[...], preferred_element_type=jnp.float32)
           + jnp.dot(mean, wn_ref[...], preferred_element_type=jnp.float32)
           + b_ref[...])
    out_ref[...] = jnp.maximum(out, 0.0)
    rd_ref[...] = jnp.broadcast_to(rd, (_BLK, 16))


def _make_tc_body(relu):
    def _body(h_ref, acc_ref, rd_ref, ws_ref, wn_ref, b_ref, out_ref):
        a = jnp.concatenate([acc_ref[0], acc_ref[1]], axis=-1)
        mean = a * rd_ref[:, 0:1]
        out = (jnp.dot(h_ref[...], ws_ref[...], preferred_element_type=jnp.float32)
               + jnp.dot(mean, wn_ref[...], preferred_element_type=jnp.float32)
               + b_ref[...])
        out_ref[...] = jnp.maximum(out, 0.0) if relu else out
    return _body


_W_SPEC = pl.BlockSpec((_D, _D), lambda i: (0, 0))
_B_SPEC = pl.BlockSpec((1, _D), lambda i: (0, 0))
_H_SPEC = pl.BlockSpec((_BLK, _D), lambda i: (i, 0))
_ACC_SPEC = pl.BlockSpec((_NC, _BLK, _DH), lambda i: (0, i, 0))
_RD_SPEC = pl.BlockSpec((_BLK, 16), lambda i: (i, 0))

_tc_layer1 = pl.pallas_call(
    _tc1_body,
    grid=(_G,),
    in_specs=[_H_SPEC, _ACC_SPEC, _RD_SPEC, _W_SPEC, _W_SPEC, _B_SPEC],
    out_specs=[_H_SPEC, _RD_SPEC],
    out_shape=[jax.ShapeDtypeStruct((_NP, _D), jnp.float32),
               jax.ShapeDtypeStruct((_NP, 16), jnp.float32)],
)

_tc_layer_relu = pl.pallas_call(
    _make_tc_body(True),
    grid=(_G,),
    in_specs=[_H_SPEC, _ACC_SPEC, _RD_SPEC, _W_SPEC, _W_SPEC, _B_SPEC],
    out_specs=_H_SPEC,
    out_shape=jax.ShapeDtypeStruct((_NP, _D), jnp.float32),
)

_tc_layer_lin = pl.pallas_call(
    _make_tc_body(False),
    grid=(_G,),
    in_specs=[_H_SPEC, _ACC_SPEC, _RD_SPEC, _W_SPEC, _W_SPEC, _B_SPEC],
    out_specs=_H_SPEC,
    out_shape=jax.ShapeDtypeStruct((_NP, _D), jnp.float32),
)


def kernel(x, edge_index, edge_attr,
           Wself1, Wneigh1, b1, Wself2, Wneigh2, b2, Wself3, Wneigh3, b3,
           Wa, ba, W2a, b2a, Wb, bb, W2b, b2b):
    src = edge_index[0]
    dst = edge_index[1]
    pad = _EPAD - _E
    srcp = jnp.concatenate(
        [src, jnp.zeros((pad,), jnp.int32)]).reshape(_NS, _NB, _B)
    dstp = jnp.concatenate(
        [dst, jnp.full((pad,), _DUMMY, jnp.int32)]).reshape(_NS, _NB, _B)
    srcp2 = jnp.stack([2 * srcp, 2 * srcp + 1])
    xp = jnp.pad(x, ((0, _NP - _N), (0, 0)))

    acc1, deg = _sc_agg_deg(xp.reshape(2 * _NP, _DH), srcp2, dstp)
    h1, rdeg = _tc_layer1(xp, acc1, deg, Wself1, Wneigh1, b1.reshape(1, _D))
    acc2 = _sc_agg(h1.reshape(2 * _NP, _DH), srcp2, dstp)
    h2 = _tc_layer_relu(h1, acc2, rdeg, Wself2, Wneigh2, b2.reshape(1, _D))
    acc3 = _sc_agg(h2.reshape(2 * _NP, _DH), srcp2, dstp)
    h3 = _tc_layer_lin(h2, acc3, rdeg, Wself3, Wneigh3, b3.reshape(1, _D))
    return h3[:_N]
